# baseline (speedup 1.0000x reference)
import functools

import jax
import jax.numpy as jnp
from jax.experimental import pallas as pl
from jax.experimental.pallas import tpu as pltpu

BRANCH1_DIMS = [270, 200, 100, 256, 384, 384, 256, 250, 250, 200]
BRANCH2_DIMS = [3, 6, 12, 24, 48, 64, 72, 96, 96, 100]
OUT_OUT = 3

N1 = len(BRANCH1_DIMS) - 1
N2 = len(BRANCH2_DIMS) - 1
LANE = 128


def _rup(n, m):
    return (n + m - 1) // m * m


P1 = [_rup(d, LANE) for d in BRANCH1_DIMS]
P2 = [_rup(d, LANE) for d in BRANCH2_DIMS]
M2 = [_rup(d, 8) for d in BRANCH2_DIMS]
P_OUT = _rup(OUT_OUT, LANE)

W1_ROW_OFF = []
_off = 0
for _l in range(N1):
    W1_ROW_OFF.append(_off)
    _off += P1[_l]
W1_ROWS = _off

B_ROW_STRIDE = 8


def _fwd_kernel(x1_ref, x2t_ref, w0_ref, w1_ref, w2t_ref, wh_ref, b_ref,
                b2t_ref, wbt_ref, o_ref, *, n_chunks):
    f32 = jnp.float32
    tb = x1_ref.shape[0]
    cs = tb // n_chunks

    h1 = [x1_ref[c * cs:(c + 1) * cs, :] for c in range(n_chunks)]
    h2t = x2t_ref[...]

    w0 = w0_ref[...]
    b0 = b_ref[0:1, 0:P1[1]]
    for c in range(n_chunks):
        h1[c] = jnp.maximum(
            jnp.dot(h1[c], w0, preferred_element_type=f32) + b0, 0.0)
    for l in range(1, N1):
        din, dout = P1[l], P1[l + 1]
        off = W1_ROW_OFF[l]
        w = w1_ref[off:off + din, 0:dout]
        b = b_ref[B_ROW_STRIDE * l:B_ROW_STRIDE * l + 1, 0:dout]
        for c in range(n_chunks):
            h1[c] = jnp.maximum(
                jnp.dot(h1[c], w, preferred_element_type=f32) + b, 0.0)

    for l in range(N2):
        mi, mo = M2[l], M2[l + 1]
        wt = w2t_ref[128 * l:128 * l + mo, 0:mi]
        bt = b2t_ref[0:mo, l:l + 1]
        h2t = jnp.maximum(
            jnp.dot(wt, h2t[0:mi, :], preferred_element_type=f32) + bt, 0.0)

    wa = wh_ref[0:P1[-1], 0:P_OUT]
    row = B_ROW_STRIDE * (N1 + N2)
    bo = b_ref[row:row + 1, 0:P_OUT]
    wbt = wbt_ref[...]
    out2t = jnp.dot(wbt, h2t, preferred_element_type=f32)
    out2 = jnp.swapaxes(out2t, 0, 1)
    for c in range(n_chunks):
        out1 = jnp.dot(h1[c], wa, preferred_element_type=f32) + bo
        o_ref[c * cs:(c + 1) * cs, :] = (
            out1[:, :OUT_OUT] + out2[c * cs:(c + 1) * cs, :OUT_OUT])


@functools.partial(jax.jit, static_argnames=("batch_tile",))
def _run(x1, x2, w1_slab, w2_slab, wh_slab, b_slab, batch_tile=256):
    B = x1.shape[0]
    tb = min(batch_tile, _rup(B, 8))
    b_pad = _rup(B, tb)

    if b_pad != B:
        x1 = jnp.zeros((b_pad, BRANCH1_DIMS[0]), jnp.float32).at[:B].set(x1)

    x2t = jnp.zeros((8, b_pad), jnp.float32).at[:BRANCH2_DIMS[0], :B].set(x2.T)
    w0 = jax.lax.slice(w1_slab, (0, 0), (BRANCH1_DIMS[0], P1[1]))
    w2t = jnp.transpose(w2_slab.reshape(N2, 128, 128), (0, 2, 1)).reshape(
        N2 * 128, 128)
    b2rows = jax.lax.slice(b_slab, (B_ROW_STRIDE * N1, 0),
                           (B_ROW_STRIDE * (N1 + N2), 128),
                           (B_ROW_STRIDE, 1))
    b2t = jnp.zeros((128, 16), jnp.float32).at[:, :N2].set(b2rows.T)
    wbt = jnp.zeros((8, M2[-1]), jnp.float32).at[:OUT_OUT, :BRANCH2_DIMS[-1]].set(
        jax.lax.slice(wh_slab, (P1[-1], 0),
                      (P1[-1] + BRANCH2_DIMS[-1], OUT_OUT)).T)

    grid = (b_pad // tb,)
    n_chunks = max(1, tb // 1024)
    out = pl.pallas_call(
        functools.partial(_fwd_kernel, n_chunks=n_chunks),
        out_shape=jax.ShapeDtypeStruct((b_pad, OUT_OUT), jnp.float32),
        grid=grid,
        in_specs=[
            pl.BlockSpec((tb, BRANCH1_DIMS[0]), lambda i: (i, 0)),
            pl.BlockSpec((8, tb), lambda i: (0, i)),
            pl.BlockSpec(w0.shape, lambda i: (0, 0)),
            pl.BlockSpec(w1_slab.shape, lambda i: (0, 0)),
            pl.BlockSpec(w2t.shape, lambda i: (0, 0)),
            pl.BlockSpec(wh_slab.shape, lambda i: (0, 0)),
            pl.BlockSpec(b_slab.shape, lambda i: (0, 0)),
            pl.BlockSpec(b2t.shape, lambda i: (0, 0)),
            pl.BlockSpec(wbt.shape, lambda i: (0, 0)),
        ],
        out_specs=pl.BlockSpec((tb, OUT_OUT), lambda i: (i, 0)),
        compiler_params=pltpu.CompilerParams(
            dimension_semantics=("parallel",)),
    )(x1, x2t, w0, w1_slab, w2t, wh_slab, b_slab, b2t, wbt)

    return out[:B] if b_pad != B else out


def kernel(x1, x2, w1_slab, w2_slab, wh_slab, b_slab):
    return _run(x1, x2, w1_slab, w2_slab, wh_slab, b_slab, batch_tile=8192)


# speedup vs baseline: 1.0074x; 1.0074x over previous
import functools

import jax
import jax.numpy as jnp
from jax.experimental import pallas as pl
from jax.experimental.pallas import tpu as pltpu

BRANCH1_DIMS = [270, 200, 100, 256, 384, 384, 256, 250, 250, 200]
BRANCH2_DIMS = [3, 6, 12, 24, 48, 64, 72, 96, 96, 100]
OUT_OUT = 3

N1 = len(BRANCH1_DIMS) - 1
N2 = len(BRANCH2_DIMS) - 1
LANE = 128


def _rup(n, m):
    return (n + m - 1) // m * m


P1 = [_rup(d, LANE) for d in BRANCH1_DIMS]
P2 = [_rup(d, LANE) for d in BRANCH2_DIMS]
M2 = [_rup(d, 8) for d in BRANCH2_DIMS]
P_OUT = _rup(OUT_OUT, LANE)

W1_ROW_OFF = []
_off = 0
for _l in range(N1):
    W1_ROW_OFF.append(_off)
    _off += P1[_l]
W1_ROWS = _off

B_ROW_STRIDE = 8


def _fwd_kernel(x1_ref, x2t_ref, w0_ref, w1_ref, w2t_ref, wh_ref, b_ref,
                b2t_ref, wbt_ref, o_ref, *, n_chunks):
    f32 = jnp.float32
    tb = x1_ref.shape[0]
    cs = tb // n_chunks

    h1 = [x1_ref[c * cs:(c + 1) * cs, :] for c in range(n_chunks)]
    h2t = [x2t_ref[:, c * cs:(c + 1) * cs] for c in range(n_chunks)]

    w0 = w0_ref[...]
    b0 = b_ref[0:1, 0:P1[1]]
    for c in range(n_chunks):
        h1[c] = jnp.maximum(
            jnp.dot(h1[c], w0, preferred_element_type=f32) + b0, 0.0)
    for l in range(1, N1):
        din, dout = P1[l], P1[l + 1]
        off = W1_ROW_OFF[l]
        w = w1_ref[off:off + din, 0:dout]
        b = b_ref[B_ROW_STRIDE * l:B_ROW_STRIDE * l + 1, 0:dout]
        for c in range(n_chunks):
            h1[c] = jnp.maximum(
                jnp.dot(h1[c], w, preferred_element_type=f32) + b, 0.0)

    for l in range(N2):
        mi, mo = M2[l], M2[l + 1]
        wt = w2t_ref[128 * l:128 * l + mo, 0:mi]
        bt = b2t_ref[0:mo, l:l + 1]
        for c in range(n_chunks):
            h2t[c] = jnp.maximum(
                jnp.dot(wt, h2t[c][0:mi, :], preferred_element_type=f32) + bt,
                0.0)

    wa = wh_ref[0:P1[-1], 0:P_OUT]
    row = B_ROW_STRIDE * (N1 + N2)
    bo = b_ref[row:row + 1, 0:P_OUT]
    wbt = wbt_ref[...]
    for c in range(n_chunks):
        out2t = jnp.dot(wbt, h2t[c], preferred_element_type=f32)
        out1 = jnp.dot(h1[c], wa, preferred_element_type=f32) + bo
        out2 = jnp.swapaxes(out2t, 0, 1)
        o_ref[c * cs:(c + 1) * cs, :] = (out1[:, :OUT_OUT]
                                         + out2[:, :OUT_OUT])


@functools.partial(jax.jit, static_argnames=("batch_tile",))
def _run(x1, x2, w1_slab, w2_slab, wh_slab, b_slab, batch_tile=256):
    B = x1.shape[0]
    tb = min(batch_tile, _rup(B, 8))
    b_pad = _rup(B, tb)

    if b_pad != B:
        x1 = jnp.zeros((b_pad, BRANCH1_DIMS[0]), jnp.float32).at[:B].set(x1)

    x2t = jnp.zeros((8, b_pad), jnp.float32).at[:BRANCH2_DIMS[0], :B].set(x2.T)
    w0 = jax.lax.slice(w1_slab, (0, 0), (BRANCH1_DIMS[0], P1[1]))
    w2t = jnp.transpose(w2_slab.reshape(N2, 128, 128), (0, 2, 1)).reshape(
        N2 * 128, 128)
    b2rows = jax.lax.slice(b_slab, (B_ROW_STRIDE * N1, 0),
                           (B_ROW_STRIDE * (N1 + N2), 128),
                           (B_ROW_STRIDE, 1))
    b2t = jnp.zeros((128, 16), jnp.float32).at[:, :N2].set(b2rows.T)
    wbt = jnp.zeros((8, M2[-1]), jnp.float32).at[:OUT_OUT, :BRANCH2_DIMS[-1]].set(
        jax.lax.slice(wh_slab, (P1[-1], 0),
                      (P1[-1] + BRANCH2_DIMS[-1], OUT_OUT)).T)

    grid = (b_pad // tb,)
    n_chunks = max(1, tb // 2048)
    out = pl.pallas_call(
        functools.partial(_fwd_kernel, n_chunks=n_chunks),
        out_shape=jax.ShapeDtypeStruct((b_pad, OUT_OUT), jnp.float32),
        grid=grid,
        in_specs=[
            pl.BlockSpec((tb, BRANCH1_DIMS[0]), lambda i: (i, 0)),
            pl.BlockSpec((8, tb), lambda i: (0, i)),
            pl.BlockSpec(w0.shape, lambda i: (0, 0)),
            pl.BlockSpec(w1_slab.shape, lambda i: (0, 0)),
            pl.BlockSpec(w2t.shape, lambda i: (0, 0)),
            pl.BlockSpec(wh_slab.shape, lambda i: (0, 0)),
            pl.BlockSpec(b_slab.shape, lambda i: (0, 0)),
            pl.BlockSpec(b2t.shape, lambda i: (0, 0)),
            pl.BlockSpec(wbt.shape, lambda i: (0, 0)),
        ],
        out_specs=pl.BlockSpec((tb, OUT_OUT), lambda i: (i, 0)),
        compiler_params=pltpu.CompilerParams(
            dimension_semantics=("parallel",)),
    )(x1, x2t, w0, w1_slab, w2t, wh_slab, b_slab, b2t, wbt)

    return out[:B] if b_pad != B else out


def kernel(x1, x2, w1_slab, w2_slab, wh_slab, b_slab):
    return _run(x1, x2, w1_slab, w2_slab, wh_slab, b_slab, batch_tile=8192)
